# baseline (speedup 1.0000x reference)
import jax
import jax.numpy as jnp
from jax.experimental import pallas as pl
from jax.experimental.pallas import tpu as pltpu


def _appnp_block(alpha_ref, adj1_ref, adj2_ref, x_ref, x0_ref, out_ref):
    a = alpha_ref[0]
    xb = x_ref[...].astype(jnp.bfloat16)
    p1 = jnp.dot(adj1_ref[0].astype(jnp.bfloat16), xb,
                 preferred_element_type=jnp.float32)
    p2 = jnp.dot(adj2_ref[0].astype(jnp.bfloat16), xb,
                 preferred_element_type=jnp.float32)
    out_ref[0] = a * p1 + (1.0 - a) * x0_ref[0]
    out_ref[1] = a * p2 + (1.0 - a) * x0_ref[1]


def kernel(x, adj, x_0, alpha):
    N, d = x.shape
    BM = 256
    H = N // 2
    adj3 = adj.reshape(2, H, N)
    x03 = x_0.reshape(2, H, d)
    out = pl.pallas_call(
        _appnp_block,
        grid=(pl.cdiv(H, BM),),
        in_specs=[
            pl.BlockSpec(memory_space=pltpu.SMEM),
            pl.BlockSpec((1, BM, N), lambda i: (0, i, 0)),
            pl.BlockSpec((1, BM, N), lambda i: (1, i, 0)),
            pl.BlockSpec((N, d), lambda i: (0, 0)),
            pl.BlockSpec((2, BM, d), lambda i: (0, i, 0)),
        ],
        out_specs=pl.BlockSpec((2, BM, d), lambda i: (0, i, 0)),
        out_shape=jax.ShapeDtypeStruct((2, H, d), jnp.float32),
    )(alpha, adj3, adj3, x, x03)
    return out.reshape(N, d)
